# baseline (speedup 1.0000x reference)
import jax
import jax.numpy as jnp
from jax import lax
from jax.experimental import pallas as pl
from jax.experimental.pallas import tpu as pltpu
from jax.experimental.pallas import tpu_sc as plsc

MU_ = 0.02
NBINS = 30
N_TOTAL = 400000
NWORK = 16
NBANKS = 4
CHUNK = 25088
LAST = N_TOTAL - (NWORK - 1) * CHUNK
COMMON = LAST
EXTRA = CHUNK - COMMON
HROW = 2 * NBINS * 16
HTOT = NBANKS * HROW
MAGIC = 0x5F3759DF


def _sc_hist(pred_f, targ_f, lw_f):
    mesh = plsc.VectorSubcoreMesh(core_axis_name="c", subcore_axis_name="s", num_cores=1)

    def body(pred_hbm, targ_hbm, lw_hbm, out_hbm, pred_v, targ_v, lw_v,
             hist_v, sem):
        cid = lax.axis_index("c")
        sid = lax.axis_index("s")
        wid = sid + cid * 0
        base = wid * CHUNK
        is_full = wid < NWORK - 1

        h1 = pltpu.async_copy(pred_hbm.at[pl.ds(base, COMMON)],
                              pred_v.at[pl.ds(0, COMMON)], sem)
        h2 = pltpu.async_copy(targ_hbm.at[pl.ds(base, COMMON)],
                              targ_v.at[pl.ds(0, COMMON)], sem)
        h3 = pltpu.async_copy(lw_hbm.at[pl.ds(base, COMMON)],
                              lw_v.at[pl.ds(0, COMMON)], sem)

        zeros16 = jnp.zeros((16,), jnp.float32)
        def zrow(j, carry):
            hist_v[pl.ds(j * 16, 16)] = zeros16
            return carry
        lax.fori_loop(0, HTOT // 16, zrow, 0)

        @pl.when(is_full)
        def _():
            pltpu.sync_copy(pred_hbm.at[pl.ds(base + COMMON, EXTRA)],
                            pred_v.at[pl.ds(COMMON, EXTRA)])
            pltpu.sync_copy(targ_hbm.at[pl.ds(base + COMMON, EXTRA)],
                            targ_v.at[pl.ds(COMMON, EXTRA)])
            pltpu.sync_copy(lw_hbm.at[pl.ds(base + COMMON, EXTRA)],
                            lw_v.at[pl.ds(COMMON, EXTRA)])

        h1.wait()
        h2.wait()
        h3.wait()

        lane = lax.iota(jnp.int32, 16)
        ones = jnp.ones((16,), jnp.float32)
        mu2 = jnp.float32(MU_ * MU_)
        mu = jnp.float32(MU_)
        c15 = jnp.float32(1.5)
        nsteps = jnp.where(is_full, CHUNK // 16, COMMON // 16)

        @plsc.parallel_loop(0, nsteps, 1, unroll=NBANKS)
        def _loop(i):
            off = pl.multiple_of(i * 16, 16)
            bank = lax.rem(i, NBANKS) * HROW
            p = pred_v[pl.ds(off, 16)]
            t = targ_v[pl.ds(off, 16)]
            w = lw_v[pl.ds(off, 16)]
            d = p - t
            q = d * d + mu2
            half = q * jnp.float32(0.5)
            ib = MAGIC - lax.shift_right_logical(
                lax.bitcast_convert_type(q, jnp.int32), 1)
            y = lax.bitcast_convert_type(ib, jnp.float32)
            y = y * (c15 - half * y * y)
            y = y * (c15 - half * y * y)
            y = y * (c15 - half * y * y)
            loss = q * y - mu
            g = jnp.abs(d) * y
            b = lax.convert_element_type(g * jnp.float32(NBINS),
                                         jnp.int32)
            b = jnp.minimum(b, NBINS - 1)
            valid = w > jnp.float32(0.0)
            addr = b * 16 + lane + bank
            plsc.addupdate_scatter(hist_v, [addr], ones, mask=valid)
            plsc.addupdate_scatter(hist_v, [addr + NBINS * 16], loss,
                                   mask=valid)
        pltpu.sync_copy(hist_v, out_hbm.at[wid])

    run = pl.kernel(
        body,
        mesh=mesh,
        out_type=jax.ShapeDtypeStruct((NWORK, HTOT), jnp.float32),
        scratch_types=[
            pltpu.VMEM((CHUNK,), jnp.float32),
            pltpu.VMEM((CHUNK,), jnp.float32),
            pltpu.VMEM((CHUNK,), jnp.float32),
            pltpu.VMEM((HTOT,), jnp.float32),
            pltpu.SemaphoreType.DMA,
        ],
        compiler_params=pltpu.CompilerParams(needs_layout_passes=False),
    )
    return run(pred_f, targ_f, lw_f)


def _final_kernel(hist_ref, out_ref):
    x = hist_ref[...]
    s = jnp.sum(x, axis=0)
    cnt = jnp.sum(s[:NBINS, :], axis=1, keepdims=True)
    lsum = jnp.sum(s[NBINS:, :], axis=1, keepdims=True)
    nz = cnt > 0.0
    n = jnp.sum(nz.astype(jnp.float32))
    r = jnp.sum(jnp.where(nz, lsum / jnp.maximum(cnt, 1.0), 0.0))
    out_ref[0, 0] = jnp.where(n > 0.0, r / jnp.maximum(n, 1.0), 0.0)


def kernel(pred, target, label_weight):
    pred_f = jnp.reshape(jnp.transpose(pred), (N_TOTAL,))
    targ_f = jnp.reshape(jnp.transpose(target), (N_TOTAL,))
    lw_f = jnp.reshape(jnp.transpose(label_weight), (N_TOTAL,))
    hist = _sc_hist(pred_f, targ_f, lw_f)
    hist = jnp.reshape(hist, (NWORK * NBANKS, 2 * NBINS, 16))
    res = pl.pallas_call(
        _final_kernel,
        out_shape=jax.ShapeDtypeStruct((1, 1), jnp.float32),
        out_specs=pl.BlockSpec(memory_space=pltpu.SMEM),
    )(hist)
    return res[0, 0]


# speedup vs baseline: 1.0124x; 1.0124x over previous
import jax
import jax.numpy as jnp
from jax import lax
from jax.experimental import pallas as pl
from jax.experimental.pallas import tpu as pltpu
from jax.experimental.pallas import tpu_sc as plsc

MU_ = 0.02
NBINS = 30
N_TOTAL = 400000
NWORK = 16
NBANKS = 4
CHUNK = 25088
LAST = N_TOTAL - (NWORK - 1) * CHUNK
COMMON = LAST
EXTRA = CHUNK - COMMON
HROW = 2 * NBINS * 16
HTOT = NBANKS * HROW
MAGIC = 0x5F3759DF


def _sc_ghmr(pred_f, targ_f, lw_f):
    mesh = plsc.VectorSubcoreMesh(core_axis_name="c", subcore_axis_name="s",
                                  num_cores=1)

    def body(pred_hbm, targ_hbm, lw_hbm, out_hbm, pred_v, targ_v, lw_v,
             hist_v, part_v, shared_v, acc_v, sem):
        sid = lax.axis_index("s")
        wid = sid
        base = wid * CHUNK
        is_full = wid < NWORK - 1

        h1 = pltpu.async_copy(pred_hbm.at[pl.ds(base, COMMON)],
                              pred_v.at[pl.ds(0, COMMON)], sem)
        h2 = pltpu.async_copy(targ_hbm.at[pl.ds(base, COMMON)],
                              targ_v.at[pl.ds(0, COMMON)], sem)
        h3 = pltpu.async_copy(lw_hbm.at[pl.ds(base, COMMON)],
                              lw_v.at[pl.ds(0, COMMON)], sem)

        zeros16 = jnp.zeros((16,), jnp.float32)
        def zrow(j, carry):
            hist_v[pl.ds(j * 16, 16)] = zeros16
            return carry
        lax.fori_loop(0, HTOT // 16, zrow, 0)

        @pl.when(is_full)
        def _():
            pltpu.sync_copy(pred_hbm.at[pl.ds(base + COMMON, EXTRA)],
                            pred_v.at[pl.ds(COMMON, EXTRA)])
            pltpu.sync_copy(targ_hbm.at[pl.ds(base + COMMON, EXTRA)],
                            targ_v.at[pl.ds(COMMON, EXTRA)])
            pltpu.sync_copy(lw_hbm.at[pl.ds(base + COMMON, EXTRA)],
                            lw_v.at[pl.ds(COMMON, EXTRA)])

        h1.wait()
        h2.wait()
        h3.wait()

        lane = lax.iota(jnp.int32, 16)
        ones = jnp.ones((16,), jnp.float32)
        mu2 = jnp.float32(MU_ * MU_)
        mu = jnp.float32(MU_)
        c15 = jnp.float32(1.5)
        nsteps = jnp.where(is_full, CHUNK // 16, COMMON // 16)

        @plsc.parallel_loop(0, nsteps, 1, unroll=NBANKS)
        def _loop(i):
            off = pl.multiple_of(i * 16, 16)
            bank = lax.rem(i, NBANKS) * HROW
            p = pred_v[pl.ds(off, 16)]
            t = targ_v[pl.ds(off, 16)]
            w = lw_v[pl.ds(off, 16)]
            d = p - t
            q = d * d + mu2
            half = q * jnp.float32(0.5)
            ib = MAGIC - lax.shift_right_logical(
                lax.bitcast_convert_type(q, jnp.int32), 1)
            y = lax.bitcast_convert_type(ib, jnp.float32)
            y = y * (c15 - half * y * y)
            y = y * (c15 - half * y * y)
            y = y * (c15 - half * y * y)
            loss = q * y - mu
            g = jnp.abs(d) * y
            b = lax.convert_element_type(g * jnp.float32(NBINS),
                                         jnp.int32)
            b = jnp.minimum(b, NBINS - 1)
            valid = w > jnp.float32(0.0)
            addr = b * 16 + lane + bank
            plsc.addupdate_scatter(hist_v, [addr], ones, mask=valid)
            plsc.addupdate_scatter(hist_v, [addr + NBINS * 16], loss,
                                   mask=valid)

        def fold(r, carry):
            acc = (hist_v[pl.ds(r * 16, 16)]
                   + hist_v[pl.ds(r * 16 + HROW, 16)]
                   + hist_v[pl.ds(r * 16 + 2 * HROW, 16)]
                   + hist_v[pl.ds(r * 16 + 3 * HROW, 16)])
            part_v[pl.ds(r * 16, 16)] = acc
            return carry
        lax.fori_loop(0, 2 * NBINS, fold, 0)
        pltpu.sync_copy(part_v, shared_v.at[sid])
        plsc.subcore_barrier()

        @pl.when(sid == 0)
        def _():
            pltpu.sync_copy(shared_v, acc_v)
            def red(r, carry):
                def red_w(wk, a):
                    return a + acc_v[wk, pl.ds(r * 16, 16)]
                tot = lax.fori_loop(1, NWORK, red_w,
                                    acc_v[0, pl.ds(r * 16, 16)])
                acc_v[0, pl.ds(r * 16, 16)] = tot
                return carry
            lax.fori_loop(0, 2 * NBINS, red, 0)

            vone = jnp.ones((16,), jnp.float32)
            vzero = jnp.zeros((16,), jnp.float32)
            def fin(bb, carry):
                n_vec, r_vec = carry
                c = jnp.sum(acc_v[0, pl.ds(bb * 16, 16)])
                l = jnp.sum(acc_v[0, pl.ds((NBINS + bb) * 16, 16)])
                cbv = vone * c
                lbv = vone * l
                nzv = cbv > jnp.float32(0.0)
                n_vec = n_vec + jnp.where(nzv, vone, vzero)
                r_vec = r_vec + jnp.where(
                    nzv, lbv / jnp.maximum(cbv, vone), vzero)
                return (n_vec, r_vec)
            n_vec, r_vec = lax.fori_loop(0, NBINS, fin, (vzero, vzero))
            res_vec = jnp.where(n_vec > jnp.float32(0.0),
                                r_vec / jnp.maximum(n_vec, vone), vzero)
            part_v[pl.ds(0, 16)] = res_vec
            pltpu.sync_copy(part_v.at[pl.ds(0, 16)], out_hbm)

    run = pl.kernel(
        body,
        mesh=mesh,
        out_type=jax.ShapeDtypeStruct((16,), jnp.float32),
        scratch_types=[
            pltpu.VMEM((CHUNK,), jnp.float32),
            pltpu.VMEM((CHUNK,), jnp.float32),
            pltpu.VMEM((CHUNK,), jnp.float32),
            pltpu.VMEM((HTOT,), jnp.float32),
            pltpu.VMEM((2 * NBINS * 16,), jnp.float32),
            pltpu.VMEM_SHARED((NWORK, 2 * NBINS * 16), jnp.float32),
            pltpu.VMEM((NWORK, 2 * NBINS * 16), jnp.float32),
            pltpu.SemaphoreType.DMA,
        ],
        compiler_params=pltpu.CompilerParams(needs_layout_passes=False),
    )
    return run(pred_f, targ_f, lw_f)


def kernel(pred, target, label_weight):
    pred_f = jnp.reshape(jnp.transpose(pred), (N_TOTAL,))
    targ_f = jnp.reshape(jnp.transpose(target), (N_TOTAL,))
    lw_f = jnp.reshape(jnp.transpose(label_weight), (N_TOTAL,))
    out = _sc_ghmr(pred_f, targ_f, lw_f)
    return out[0]


# speedup vs baseline: 1.0780x; 1.0649x over previous
import jax
import jax.numpy as jnp
from jax import lax
from jax.experimental import pallas as pl
from jax.experimental.pallas import tpu as pltpu
from jax.experimental.pallas import tpu_sc as plsc

MU_ = 0.02
NBINS = 30
N_TOTAL = 400000
NWORK = 16
NBANKS = 4
CHUNK = 25088
LAST = N_TOTAL - (NWORK - 1) * CHUNK
COMMON = LAST
EXTRA = CHUNK - COMMON
HROW = 2 * NBINS * 16
HTOT = NBANKS * HROW
MAGIC = 0x5F3759DF


def _sc_ghmr(pred_f, targ_f, lw_f):
    mesh = plsc.VectorSubcoreMesh(core_axis_name="c", subcore_axis_name="s",
                                  num_cores=1)

    def body(pred_hbm, targ_hbm, lw_hbm, out_hbm, pred_v, targ_v, lw_v,
             hist_v, part_v, shared_v, acc_v, sem):
        sid = lax.axis_index("s")
        wid = sid
        base = wid * CHUNK
        is_full = wid < NWORK - 1

        h1 = pltpu.async_copy(pred_hbm.at[pl.ds(base, COMMON)],
                              pred_v.at[pl.ds(0, COMMON)], sem)
        h2 = pltpu.async_copy(targ_hbm.at[pl.ds(base, COMMON)],
                              targ_v.at[pl.ds(0, COMMON)], sem)
        h3 = pltpu.async_copy(lw_hbm.at[pl.ds(base, COMMON)],
                              lw_v.at[pl.ds(0, COMMON)], sem)

        zeros16 = jnp.zeros((16,), jnp.float32)
        def zrow(j, carry):
            hist_v[pl.ds(j * 16, 16)] = zeros16
            return carry
        lax.fori_loop(0, HTOT // 16, zrow, 0)

        @pl.when(is_full)
        def _():
            pltpu.sync_copy(pred_hbm.at[pl.ds(base + COMMON, EXTRA)],
                            pred_v.at[pl.ds(COMMON, EXTRA)])
            pltpu.sync_copy(targ_hbm.at[pl.ds(base + COMMON, EXTRA)],
                            targ_v.at[pl.ds(COMMON, EXTRA)])
            pltpu.sync_copy(lw_hbm.at[pl.ds(base + COMMON, EXTRA)],
                            lw_v.at[pl.ds(COMMON, EXTRA)])

        h1.wait()
        h2.wait()
        h3.wait()

        lane = lax.iota(jnp.int32, 16)
        ones = jnp.ones((16,), jnp.float32)
        mu2 = jnp.float32(MU_ * MU_)
        mu = jnp.float32(MU_)
        c15 = jnp.float32(1.5)
        nsteps = jnp.where(is_full, CHUNK // 16, COMMON // 16)

        @plsc.parallel_loop(0, nsteps, 1, unroll=NBANKS)
        def _loop(i):
            off = pl.multiple_of(i * 16, 16)
            bank = lax.rem(i, NBANKS) * HROW
            p = pred_v[pl.ds(off, 16)]
            t = targ_v[pl.ds(off, 16)]
            w = lw_v[pl.ds(off, 16)]
            d = p - t
            q = d * d + mu2
            half = q * jnp.float32(0.5)
            ib = MAGIC - lax.shift_right_logical(
                lax.bitcast_convert_type(q, jnp.int32), 1)
            y = lax.bitcast_convert_type(ib, jnp.float32)
            y = y * (c15 - half * y * y)
            y = y * (c15 - half * y * y)
            y = y * (c15 - half * y * y)
            loss = q * y - mu
            g = jnp.abs(d) * y
            b = lax.convert_element_type(g * jnp.float32(NBINS),
                                         jnp.int32)
            b = jnp.minimum(b, NBINS - 1)
            valid = w > jnp.float32(0.0)
            addr = b * 16 + lane + bank
            plsc.addupdate_scatter(hist_v, [addr], ones, mask=valid)
            plsc.addupdate_scatter(hist_v, [addr + NBINS * 16], loss,
                                   mask=valid)

        plsc.subcore_barrier()

        def fold(r, carry):
            acc = (hist_v[pl.ds(r * 16, 16)]
                   + hist_v[pl.ds(r * 16 + HROW, 16)]
                   + hist_v[pl.ds(r * 16 + 2 * HROW, 16)]
                   + hist_v[pl.ds(r * 16 + 3 * HROW, 16)])
            part_v[pl.ds(r * 16, 16)] = acc
            return carry
        lax.fori_loop(0, 2 * NBINS, fold, 0)
        pltpu.sync_copy(part_v, shared_v.at[pl.ds(sid * HROW, HROW)])
        plsc.subcore_barrier()

        @pl.when(sid == 0)
        def _():
            pltpu.sync_copy(shared_v, acc_v)
            def red(r, carry):
                def red_w(wk, a):
                    return a + acc_v[pl.ds(wk * HROW + r * 16, 16)]
                tot = lax.fori_loop(1, NWORK, red_w,
                                    acc_v[pl.ds(r * 16, 16)])
                acc_v[pl.ds(r * 16, 16)] = tot
                return carry
            lax.fori_loop(0, 2 * NBINS, red, 0)

            vone = jnp.ones((16,), jnp.float32)
            vzero = jnp.zeros((16,), jnp.float32)
            def fin(bb, carry):
                n_vec, r_vec = carry
                c = jnp.sum(acc_v[pl.ds(bb * 16, 16)])
                l = jnp.sum(acc_v[pl.ds((NBINS + bb) * 16, 16)])
                cbv = vone * c
                lbv = vone * l
                nzv = cbv > jnp.float32(0.0)
                n_vec = n_vec + jnp.where(nzv, vone, vzero)
                r_vec = r_vec + jnp.where(
                    nzv, lbv / jnp.maximum(cbv, vone), vzero)
                return (n_vec, r_vec)
            n_vec, r_vec = lax.fori_loop(0, NBINS, fin, (vzero, vzero))
            res_vec = jnp.where(n_vec > jnp.float32(0.0),
                                r_vec / jnp.maximum(n_vec, vone), vzero)
            part_v[pl.ds(0, 16)] = res_vec
            pltpu.sync_copy(part_v.at[pl.ds(0, 16)], out_hbm)


    run = pl.kernel(
        body,
        mesh=mesh,
        out_type=jax.ShapeDtypeStruct((16,), jnp.float32),
        scratch_types=[
            pltpu.VMEM((CHUNK,), jnp.float32),
            pltpu.VMEM((CHUNK,), jnp.float32),
            pltpu.VMEM((CHUNK,), jnp.float32),
            pltpu.VMEM((HTOT,), jnp.float32),
            pltpu.VMEM((2 * NBINS * 16,), jnp.float32),
            pltpu.VMEM_SHARED((NWORK * 2 * NBINS * 16,), jnp.float32),
            pltpu.VMEM((NWORK * 2 * NBINS * 16,), jnp.float32),
            pltpu.SemaphoreType.DMA,
        ],
        compiler_params=pltpu.CompilerParams(needs_layout_passes=False),
    )
    return run(pred_f, targ_f, lw_f)


def kernel(pred, target, label_weight):
    pred_f = jnp.reshape(jnp.transpose(pred), (N_TOTAL,))
    targ_f = jnp.reshape(jnp.transpose(target), (N_TOTAL,))
    lw_f = jnp.reshape(jnp.transpose(label_weight), (N_TOTAL,))
    out = _sc_ghmr(pred_f, targ_f, lw_f)
    return out[0]


# speedup vs baseline: 1.1104x; 1.0300x over previous
import jax
import jax.numpy as jnp
from jax import lax
from jax.experimental import pallas as pl
from jax.experimental.pallas import tpu as pltpu
from jax.experimental.pallas import tpu_sc as plsc

MU_ = 0.02
NBINS = 30
N_TOTAL = 400000
NWORK = 16
NBANKS = 4
CHUNK = 25088
LAST = N_TOTAL - (NWORK - 1) * CHUNK
COMMON = LAST
EXTRA = CHUNK - COMMON
HROW = 2 * NBINS * 16
HTOT = NBANKS * HROW
MAGIC = 0x5F3759DF


def _sc_ghmr(pred_f, targ_f, lw_f):
    mesh = plsc.VectorSubcoreMesh(core_axis_name="c", subcore_axis_name="s",
                                  num_cores=1)

    def body(pred_hbm, targ_hbm, lw_hbm, out_hbm, pred_v, targ_v, lw_v,
             hist_v, part_v, shared_v, acc_v, sem):
        sid = lax.axis_index("s")
        wid = sid
        base = wid * CHUNK
        is_full = wid < NWORK - 1

        h1 = pltpu.async_copy(pred_hbm.at[pl.ds(base, COMMON)],
                              pred_v.at[pl.ds(0, COMMON)], sem)
        h2 = pltpu.async_copy(targ_hbm.at[pl.ds(base, COMMON)],
                              targ_v.at[pl.ds(0, COMMON)], sem)
        h3 = pltpu.async_copy(lw_hbm.at[pl.ds(base, COMMON)],
                              lw_v.at[pl.ds(0, COMMON)], sem)

        zeros16 = jnp.zeros((16,), jnp.float32)
        def zrow(j, carry):
            hist_v[pl.ds(j * 16, 16)] = zeros16
            return carry
        lax.fori_loop(0, HTOT // 16, zrow, 0)

        @pl.when(is_full)
        def _():
            pltpu.sync_copy(pred_hbm.at[pl.ds(base + COMMON, EXTRA)],
                            pred_v.at[pl.ds(COMMON, EXTRA)])
            pltpu.sync_copy(targ_hbm.at[pl.ds(base + COMMON, EXTRA)],
                            targ_v.at[pl.ds(COMMON, EXTRA)])
            pltpu.sync_copy(lw_hbm.at[pl.ds(base + COMMON, EXTRA)],
                            lw_v.at[pl.ds(COMMON, EXTRA)])

        h1.wait()
        h2.wait()
        h3.wait()

        lane = lax.iota(jnp.int32, 16)
        ones = jnp.ones((16,), jnp.float32)
        mu2 = jnp.float32(MU_ * MU_)
        mu = jnp.float32(MU_)
        c15 = jnp.float32(1.5)
        nsteps = jnp.where(is_full, CHUNK // 16, COMMON // 16)

        @plsc.parallel_loop(0, nsteps, 1, unroll=NBANKS)
        def _loop(i):
            off = pl.multiple_of(i * 16, 16)
            bank = lax.rem(i, NBANKS) * HROW
            p = pred_v[pl.ds(off, 16)]
            t = targ_v[pl.ds(off, 16)]
            w = lw_v[pl.ds(off, 16)]
            d = p - t
            q = d * d + mu2
            half = q * jnp.float32(0.5)
            ib = MAGIC - lax.shift_right_logical(
                lax.bitcast_convert_type(q, jnp.int32), 1)
            y = lax.bitcast_convert_type(ib, jnp.float32)
            y = y * (c15 - half * y * y)
            y = y * (c15 - half * y * y)
            loss = q * y - mu
            g = jnp.abs(d) * y
            b = lax.convert_element_type(g * jnp.float32(NBINS),
                                         jnp.int32)
            b = jnp.minimum(b, NBINS - 1)
            valid = w > jnp.float32(0.0)
            addr = b * 16 + lane + bank
            plsc.addupdate_scatter(hist_v, [addr], ones, mask=valid)
            plsc.addupdate_scatter(hist_v, [addr + NBINS * 16], loss,
                                   mask=valid)

        plsc.subcore_barrier()

        def fold(r, carry):
            acc = (hist_v[pl.ds(r * 16, 16)]
                   + hist_v[pl.ds(r * 16 + HROW, 16)]
                   + hist_v[pl.ds(r * 16 + 2 * HROW, 16)]
                   + hist_v[pl.ds(r * 16 + 3 * HROW, 16)])
            part_v[pl.ds(r * 16, 16)] = acc
            return carry
        lax.fori_loop(0, 2 * NBINS, fold, 0)
        pltpu.sync_copy(part_v, shared_v.at[pl.ds(sid * HROW, HROW)])
        plsc.subcore_barrier()

        @pl.when(sid == 0)
        def _():
            pltpu.sync_copy(shared_v, acc_v)
            def red(r, carry):
                def red_w(wk, a):
                    return a + acc_v[pl.ds(wk * HROW + r * 16, 16)]
                tot = lax.fori_loop(1, NWORK, red_w,
                                    acc_v[pl.ds(r * 16, 16)])
                acc_v[pl.ds(r * 16, 16)] = tot
                return carry
            lax.fori_loop(0, 2 * NBINS, red, 0)

            vone = jnp.ones((16,), jnp.float32)
            vzero = jnp.zeros((16,), jnp.float32)
            def fin(bb, carry):
                n_vec, r_vec = carry
                c = jnp.sum(acc_v[pl.ds(bb * 16, 16)])
                l = jnp.sum(acc_v[pl.ds((NBINS + bb) * 16, 16)])
                cbv = vone * c
                lbv = vone * l
                nzv = cbv > jnp.float32(0.0)
                n_vec = n_vec + jnp.where(nzv, vone, vzero)
                r_vec = r_vec + jnp.where(
                    nzv, lbv / jnp.maximum(cbv, vone), vzero)
                return (n_vec, r_vec)
            n_vec, r_vec = lax.fori_loop(0, NBINS, fin, (vzero, vzero))
            res_vec = jnp.where(n_vec > jnp.float32(0.0),
                                r_vec / jnp.maximum(n_vec, vone), vzero)
            part_v[pl.ds(0, 16)] = res_vec
            pltpu.sync_copy(part_v.at[pl.ds(0, 16)], out_hbm)


    run = pl.kernel(
        body,
        mesh=mesh,
        out_type=jax.ShapeDtypeStruct((16,), jnp.float32),
        scratch_types=[
            pltpu.VMEM((CHUNK,), jnp.float32),
            pltpu.VMEM((CHUNK,), jnp.float32),
            pltpu.VMEM((CHUNK,), jnp.float32),
            pltpu.VMEM((HTOT,), jnp.float32),
            pltpu.VMEM((2 * NBINS * 16,), jnp.float32),
            pltpu.VMEM_SHARED((NWORK * 2 * NBINS * 16,), jnp.float32),
            pltpu.VMEM((NWORK * 2 * NBINS * 16,), jnp.float32),
            pltpu.SemaphoreType.DMA,
        ],
        compiler_params=pltpu.CompilerParams(needs_layout_passes=False),
    )
    return run(pred_f, targ_f, lw_f)


def kernel(pred, target, label_weight):
    pred_f = jnp.reshape(jnp.transpose(pred), (N_TOTAL,))
    targ_f = jnp.reshape(jnp.transpose(target), (N_TOTAL,))
    lw_f = jnp.reshape(jnp.transpose(label_weight), (N_TOTAL,))
    out = _sc_ghmr(pred_f, targ_f, lw_f)
    return out[0]
